# baseline (speedup 1.0000x reference)
import functools

import jax
import jax.numpy as jnp
from jax import lax
from jax.experimental import pallas as pl
from jax.experimental.pallas import tpu as pltpu
from jax.experimental.pallas import tpu_sc as plsc

_K = 256
_L = 16
_NB = 256
_U = 8
_SB = 1
_BANK = _NB * _L


def _sc_body(x_hbm, o_hbm, xbuf0, xbuf1, hist2, hist, sin0, sin1, sout0,
             sout1):
    cols = x_hbm.shape[1]
    ngrp = cols // (_L * _U)
    c = lax.axis_index("c")
    s = lax.axis_index("s")
    wid = s * 2 + c
    rows_per = x_hbm.shape[0] // 32

    iota = lax.iota(jnp.int32, _L)
    ones_i = jnp.ones((_L,), jnp.int32)
    zeros_i = jnp.zeros((_L,), jnp.int32)
    lane255 = iota * _NB + 255
    lane127 = iota * _NB + 127

    def zero_hist():
        def zb(i, _):
            hist2[pl.ds(i * _L, _L)] = zeros_i
            return 0

        lax.fori_loop(0, _BANK // _L, zb, 0, unroll=8)

    def zero_all_subbanks():
        def zb(i, _):
            hist2[pl.ds(_BANK + i * _L, _L)] = zeros_i
            return 0

        lax.fori_loop(0, ((_SB - 1) * _BANK) // _L, zb, 0, unroll=8)

    def fold_subbanks():
        def fb(i, _):
            acc = hist2[pl.ds(i * _L, _L)]
            for u in range(1, _SB):
                off = u * _BANK + i * _L
                acc = acc + hist2[pl.ds(off, _L)]
                hist2[pl.ds(off, _L)] = zeros_i
            hist2[pl.ds(i * _L, _L)] = acc
            return 0

        lax.fori_loop(0, _BANK // _L, fb, 0)

    def reduce_hist(nbins):
        def hr(chunk, _):
            def acc_one(l, acc):
                return acc + hist2[pl.ds(l * _NB + chunk * _L, _L)]

            acc = lax.fori_loop(0, _L, acc_one, zeros_i, unroll=4)
            hist[pl.ds(chunk * _L, _L)] = acc
            return 0

        lax.fori_loop(0, nbins // _L, hr, 0)

    def walk_hist(nbins, kr):
        def wstep(jj, carry):
            s_run, found, b_star, s_before = carry
            chunk = hist[pl.ds(jj * _L, _L)]
            cs = plsc.cumsum(chunk)
            tot = jnp.sum(chunk)
            m = (s_run + cs) >= kr
            pc = plsc.all_reduce_population_count(m)[0]
            first = plsc.all_reduce_ffs(m)[0]
            crossed = (found == 0) & (pc > 0)
            in_before = jnp.sum(jnp.where(iota < first, chunk, 0))
            b_star = jnp.where(crossed, jj * _L + first, b_star)
            s_before = jnp.where(crossed, s_run + in_before, s_before)
            found = found | (pc > 0).astype(jnp.int32)
            return (s_run + tot, found, b_star, s_before)

        init = (jnp.int32(0), jnp.int32(0), jnp.int32(0), jnp.int32(0))
        _, _, b_star, s_before = lax.fori_loop(0, nbins // _L, wstep, init)
        return b_star, s_before

    def scan_level(xbuf, shift, width, prefix):
        mbits = (1 << width) - 1
        lane_x = lane255 if width == 8 else lane127

        def grp(g, _):
            base = g * _U * _L
            abs_ = []
            for u in range(_U):
                v = xbuf[pl.ds(base + u * _L, _L)]
                b = lax.bitcast_convert_type(v, jnp.int32)
                abs_.append(b & jnp.int32(0x7FFFFFFF))
            idxs = []
            masks = []
            for ab in abs_:
                if shift:
                    t = lax.shift_right_logical(ab, shift)
                else:
                    t = ab
                y = t & jnp.int32(mbits) if (shift != 23 or width != 8) else t
                idxs.append(lane_x ^ y)
                if prefix is None:
                    masks.append(None)
                else:
                    masks.append(
                        lax.shift_right_logical(t, width) == prefix)
            for u in range(_U):
                plsc.addupdate_scatter(
                    hist2.at[pl.ds((u % _SB) * _BANK, _BANK)],
                    [idxs[u]], ones_i, mask=masks[u])
            return 0

        lax.fori_loop(0, ngrp, grp, 0)

    def process_row(xbuf):
        prefix = None
        kr = jnp.int32(_K)
        for (shift, width) in ((23, 8), (15, 8), (7, 8), (0, 7)):
            nbins = 1 << width
            zero_hist()
            scan_level(xbuf, shift, width, prefix)
            reduce_hist(nbins)
            b_star, s_before = walk_hist(nbins, kr)
            y = jnp.int32(nbins - 1) - b_star
            prefix = y if prefix is None else (
                lax.shift_left(prefix, width) | y)
            kr = kr - s_before

        thresh = prefix

        def mk(g, _):
            base = g * _U * _L
            vs = []
            for u in range(_U):
                vs.append(xbuf[pl.ds(base + u * _L, _L)])
            outs = []
            for v in vs:
                b = lax.bitcast_convert_type(v, jnp.int32)
                ab = b & jnp.int32(0x7FFFFFFF)
                outs.append(jnp.where(ab >= thresh, v, jnp.float32(0.0)))
            for u in range(_U):
                xbuf[pl.ds(base + u * _L, _L)] = outs[u]
            return 0

        lax.fori_loop(0, ngrp, mk, 0)

    bufs = (xbuf0, xbuf1)
    sins = (sin0, sin1)
    souts = (sout0, sout1)
    row0 = wid * rows_per
    nrows = 4
    in_descs = [None] * nrows
    out_descs = [None] * nrows
    in_descs[0] = pltpu.async_copy(x_hbm.at[row0], bufs[0], sins[0])
    for j in range(nrows):
        if j + 1 < nrows:
            if j >= 1:
                out_descs[j - 1].wait()
            in_descs[j + 1] = pltpu.async_copy(
                x_hbm.at[row0 + j + 1], bufs[(j + 1) % 2], sins[(j + 1) % 2])
        in_descs[j].wait()
        process_row(bufs[j % 2])
        out_descs[j] = pltpu.async_copy(bufs[j % 2], o_hbm.at[row0 + j],
                                        souts[j % 2])
    out_descs[nrows - 2].wait()
    out_descs[nrows - 1].wait()


def kernel(x, k):
    del k
    rows, cols = x.shape
    mesh = plsc.VectorSubcoreMesh(core_axis_name="c", subcore_axis_name="s")
    f = functools.partial(
        pl.kernel,
        out_type=jax.ShapeDtypeStruct((rows, cols), x.dtype),
        mesh=mesh,
        compiler_params=pltpu.CompilerParams(needs_layout_passes=False),
        scratch_types=[
            pltpu.VMEM((cols,), jnp.float32),
            pltpu.VMEM((cols,), jnp.float32),
            pltpu.VMEM((_SB * _NB * _L,), jnp.int32),
            pltpu.VMEM((_NB,), jnp.int32),
            pltpu.SemaphoreType.DMA,
            pltpu.SemaphoreType.DMA,
            pltpu.SemaphoreType.DMA,
            pltpu.SemaphoreType.DMA,
        ],
    )(_sc_body)
    return f(x)


# speedup vs baseline: 1.1248x; 1.1248x over previous
import functools

import jax
import jax.numpy as jnp
from jax import lax
from jax.experimental import pallas as pl
from jax.experimental.pallas import tpu as pltpu
from jax.experimental.pallas import tpu_sc as plsc

_K = 256
_L = 16
_NB = 256
_U = 8
_SB = 1
_BANK = _NB * _L


def _sc_body(x_hbm, o_hbm, xbuf0, xbuf1, hist2, hist, sin0, sin1, sout0,
             sout1):
    cols = x_hbm.shape[1]
    ngrp = cols // (_L * _U)
    c = lax.axis_index("c")
    s = lax.axis_index("s")
    wid = s * 2 + c
    rows_per = x_hbm.shape[0] // 32

    iota = lax.iota(jnp.int32, _L)
    ones_i = jnp.ones((_L,), jnp.int32)
    zeros_i = jnp.zeros((_L,), jnp.int32)
    lane255 = jnp.full((_L,), 255, jnp.int32)
    lane127 = jnp.full((_L,), 127, jnp.int32)

    def zero_hist():
        def zb(i, _):
            hist2[pl.ds(i * _L, _L)] = zeros_i
            return 0

        lax.fori_loop(0, _NB // _L, zb, 0, unroll=8)

    def zero_all_subbanks():
        def zb(i, _):
            hist2[pl.ds(_BANK + i * _L, _L)] = zeros_i
            return 0

        lax.fori_loop(0, ((_SB - 1) * _BANK) // _L, zb, 0, unroll=8)

    def fold_subbanks():
        def fb(i, _):
            acc = hist2[pl.ds(i * _L, _L)]
            for u in range(1, _SB):
                off = u * _BANK + i * _L
                acc = acc + hist2[pl.ds(off, _L)]
                hist2[pl.ds(off, _L)] = zeros_i
            hist2[pl.ds(i * _L, _L)] = acc
            return 0

        lax.fori_loop(0, _BANK // _L, fb, 0)

    def walk_hist(nbins, kr):
        def wstep(jj, carry):
            s_run, found, b_star, s_before = carry
            chunk = hist2[pl.ds(jj * _L, _L)]
            cs = plsc.cumsum(chunk)
            tot = jnp.sum(chunk)
            m = (s_run + cs) >= kr
            pc = plsc.all_reduce_population_count(m)[0]
            first = plsc.all_reduce_ffs(m)[0]
            crossed = (found == 0) & (pc > 0)
            in_before = jnp.sum(jnp.where(iota < first, chunk, 0))
            b_star = jnp.where(crossed, jj * _L + first, b_star)
            s_before = jnp.where(crossed, s_run + in_before, s_before)
            found = found | (pc > 0).astype(jnp.int32)
            return (s_run + tot, found, b_star, s_before)

        init = (jnp.int32(0), jnp.int32(0), jnp.int32(0), jnp.int32(0))
        _, _, b_star, s_before = lax.fori_loop(0, nbins // _L, wstep, init)
        return b_star, s_before

    def scan_level(xbuf, shift, width, prefix):
        mbits = (1 << width) - 1
        lane_x = lane255 if width == 8 else lane127

        def grp(g, _):
            base = g * _U * _L
            abs_ = []
            for u in range(_U):
                v = xbuf[pl.ds(base + u * _L, _L)]
                b = lax.bitcast_convert_type(v, jnp.int32)
                abs_.append(b & jnp.int32(0x7FFFFFFF))
            idxs = []
            masks = []
            for ab in abs_:
                if shift:
                    t = lax.shift_right_logical(ab, shift)
                else:
                    t = ab
                y = t & jnp.int32(mbits) if (shift != 23 or width != 8) else t
                idxs.append(lane_x ^ y)
                if prefix is None:
                    masks.append(None)
                else:
                    masks.append(
                        lax.shift_right_logical(t, width) == prefix)
            for u in range(_U):
                plsc.addupdate_scatter(
                    hist2.at[pl.ds((u % _SB) * _BANK, _BANK)],
                    [idxs[u]], ones_i, mask=masks[u])
            return 0

        lax.fori_loop(0, ngrp, grp, 0)

    def process_row(xbuf):
        prefix = None
        kr = jnp.int32(_K)
        for (shift, width) in ((23, 8), (15, 8), (7, 8), (0, 7)):
            nbins = 1 << width
            zero_hist()
            scan_level(xbuf, shift, width, prefix)
            b_star, s_before = walk_hist(nbins, kr)
            y = jnp.int32(nbins - 1) - b_star
            prefix = y if prefix is None else (
                lax.shift_left(prefix, width) | y)
            kr = kr - s_before

        thresh = prefix

        def mk(g, _):
            base = g * _U * _L
            vs = []
            for u in range(_U):
                vs.append(xbuf[pl.ds(base + u * _L, _L)])
            outs = []
            for v in vs:
                b = lax.bitcast_convert_type(v, jnp.int32)
                ab = b & jnp.int32(0x7FFFFFFF)
                outs.append(jnp.where(ab >= thresh, v, jnp.float32(0.0)))
            for u in range(_U):
                xbuf[pl.ds(base + u * _L, _L)] = outs[u]
            return 0

        lax.fori_loop(0, ngrp, mk, 0)

    bufs = (xbuf0, xbuf1)
    sins = (sin0, sin1)
    souts = (sout0, sout1)
    row0 = wid * rows_per
    nrows = 4
    in_descs = [None] * nrows
    out_descs = [None] * nrows
    in_descs[0] = pltpu.async_copy(x_hbm.at[row0], bufs[0], sins[0])
    for j in range(nrows):
        if j + 1 < nrows:
            if j >= 1:
                out_descs[j - 1].wait()
            in_descs[j + 1] = pltpu.async_copy(
                x_hbm.at[row0 + j + 1], bufs[(j + 1) % 2], sins[(j + 1) % 2])
        in_descs[j].wait()
        process_row(bufs[j % 2])
        out_descs[j] = pltpu.async_copy(bufs[j % 2], o_hbm.at[row0 + j],
                                        souts[j % 2])
    out_descs[nrows - 2].wait()
    out_descs[nrows - 1].wait()


def kernel(x, k):
    del k
    rows, cols = x.shape
    mesh = plsc.VectorSubcoreMesh(core_axis_name="c", subcore_axis_name="s")
    f = functools.partial(
        pl.kernel,
        out_type=jax.ShapeDtypeStruct((rows, cols), x.dtype),
        mesh=mesh,
        compiler_params=pltpu.CompilerParams(needs_layout_passes=False),
        scratch_types=[
            pltpu.VMEM((cols,), jnp.float32),
            pltpu.VMEM((cols,), jnp.float32),
            pltpu.VMEM((_NB,), jnp.int32),
            pltpu.VMEM((_NB,), jnp.int32),
            pltpu.SemaphoreType.DMA,
            pltpu.SemaphoreType.DMA,
            pltpu.SemaphoreType.DMA,
            pltpu.SemaphoreType.DMA,
        ],
    )(_sc_body)
    return f(x)


# speedup vs baseline: 1.2873x; 1.1445x over previous
import functools

import jax
import jax.numpy as jnp
from jax import lax
from jax.experimental import pallas as pl
from jax.experimental.pallas import tpu as pltpu
from jax.experimental.pallas import tpu_sc as plsc

_K = 256
_L = 16
_NB = 256
_U = 16
_SB = 1
_BANK = _NB * _L


def _sc_body(x_hbm, o_hbm, xbuf0, xbuf1, hist2, hist, sin0, sin1, sout0,
             sout1):
    cols = x_hbm.shape[1]
    ngrp = cols // (_L * _U)
    c = lax.axis_index("c")
    s = lax.axis_index("s")
    wid = s * 2 + c
    rows_per = x_hbm.shape[0] // 32

    iota = lax.iota(jnp.int32, _L)
    ones_i = jnp.ones((_L,), jnp.int32)
    zeros_i = jnp.zeros((_L,), jnp.int32)
    lane255 = jnp.full((_L,), 255, jnp.int32)
    lane127 = jnp.full((_L,), 127, jnp.int32)

    def zero_hist():
        def zb(i, _):
            hist2[pl.ds(i * _L, _L)] = zeros_i
            return 0

        lax.fori_loop(0, _NB // _L, zb, 0, unroll=8)

    def zero_all_subbanks():
        def zb(i, _):
            hist2[pl.ds(_BANK + i * _L, _L)] = zeros_i
            return 0

        lax.fori_loop(0, ((_SB - 1) * _BANK) // _L, zb, 0, unroll=8)

    def fold_subbanks():
        def fb(i, _):
            acc = hist2[pl.ds(i * _L, _L)]
            for u in range(1, _SB):
                off = u * _BANK + i * _L
                acc = acc + hist2[pl.ds(off, _L)]
                hist2[pl.ds(off, _L)] = zeros_i
            hist2[pl.ds(i * _L, _L)] = acc
            return 0

        lax.fori_loop(0, _BANK // _L, fb, 0)

    def walk_hist(nbins, kr):
        def wstep(jj, carry):
            s_run, found, b_star, s_before = carry
            chunk = hist2[pl.ds(jj * _L, _L)]
            cs = plsc.cumsum(chunk)
            tot = jnp.sum(chunk)
            m = (s_run + cs) >= kr
            pc = plsc.all_reduce_population_count(m)[0]
            first = plsc.all_reduce_ffs(m)[0]
            crossed = (found == 0) & (pc > 0)
            in_before = jnp.sum(jnp.where(iota < first, chunk, 0))
            b_star = jnp.where(crossed, jj * _L + first, b_star)
            s_before = jnp.where(crossed, s_run + in_before, s_before)
            found = found | (pc > 0).astype(jnp.int32)
            return (s_run + tot, found, b_star, s_before)

        init = (jnp.int32(0), jnp.int32(0), jnp.int32(0), jnp.int32(0))
        _, _, b_star, s_before = lax.fori_loop(0, nbins // _L, wstep, init)
        return b_star, s_before

    def scan_level(xbuf, shift, width, prefix):
        mbits = (1 << width) - 1
        lane_x = lane255 if width == 8 else lane127

        def grp(g, _):
            base = g * _U * _L
            abs_ = []
            for u in range(_U):
                v = xbuf[pl.ds(base + u * _L, _L)]
                b = lax.bitcast_convert_type(v, jnp.int32)
                abs_.append(b & jnp.int32(0x7FFFFFFF))
            idxs = []
            masks = []
            for ab in abs_:
                if shift:
                    t = lax.shift_right_logical(ab, shift)
                else:
                    t = ab
                y = t & jnp.int32(mbits) if (shift != 23 or width != 8) else t
                idxs.append(lane_x ^ y)
                if prefix is None:
                    masks.append(None)
                else:
                    masks.append(
                        lax.shift_right_logical(t, width) == prefix)
            for u in range(_U):
                plsc.addupdate_scatter(
                    hist2.at[pl.ds((u % _SB) * _BANK, _BANK)],
                    [idxs[u]], ones_i, mask=masks[u])
            return 0

        lax.fori_loop(0, ngrp, grp, 0)

    def process_row(xbuf):
        prefix = None
        kr = jnp.int32(_K)
        for (shift, width) in ((23, 8), (15, 8), (7, 8), (0, 7)):
            nbins = 1 << width
            zero_hist()
            scan_level(xbuf, shift, width, prefix)
            b_star, s_before = walk_hist(nbins, kr)
            y = jnp.int32(nbins - 1) - b_star
            prefix = y if prefix is None else (
                lax.shift_left(prefix, width) | y)
            kr = kr - s_before

        thresh = prefix

        def mk(g, _):
            base = g * _U * _L
            vs = []
            for u in range(_U):
                vs.append(xbuf[pl.ds(base + u * _L, _L)])
            outs = []
            for v in vs:
                b = lax.bitcast_convert_type(v, jnp.int32)
                ab = b & jnp.int32(0x7FFFFFFF)
                outs.append(jnp.where(ab >= thresh, v, jnp.float32(0.0)))
            for u in range(_U):
                xbuf[pl.ds(base + u * _L, _L)] = outs[u]
            return 0

        lax.fori_loop(0, ngrp, mk, 0)

    bufs = (xbuf0, xbuf1)
    sins = (sin0, sin1)
    souts = (sout0, sout1)
    row0 = wid * rows_per
    nrows = 4
    in_descs = [None] * nrows
    out_descs = [None] * nrows
    in_descs[0] = pltpu.async_copy(x_hbm.at[row0], bufs[0], sins[0])
    for j in range(nrows):
        if j + 1 < nrows:
            if j >= 1:
                out_descs[j - 1].wait()
            in_descs[j + 1] = pltpu.async_copy(
                x_hbm.at[row0 + j + 1], bufs[(j + 1) % 2], sins[(j + 1) % 2])
        in_descs[j].wait()
        process_row(bufs[j % 2])
        out_descs[j] = pltpu.async_copy(bufs[j % 2], o_hbm.at[row0 + j],
                                        souts[j % 2])
    out_descs[nrows - 2].wait()
    out_descs[nrows - 1].wait()


def kernel(x, k):
    del k
    rows, cols = x.shape
    mesh = plsc.VectorSubcoreMesh(core_axis_name="c", subcore_axis_name="s")
    f = functools.partial(
        pl.kernel,
        out_type=jax.ShapeDtypeStruct((rows, cols), x.dtype),
        mesh=mesh,
        compiler_params=pltpu.CompilerParams(needs_layout_passes=False),
        scratch_types=[
            pltpu.VMEM((cols,), jnp.float32),
            pltpu.VMEM((cols,), jnp.float32),
            pltpu.VMEM((_NB,), jnp.int32),
            pltpu.VMEM((_NB,), jnp.int32),
            pltpu.SemaphoreType.DMA,
            pltpu.SemaphoreType.DMA,
            pltpu.SemaphoreType.DMA,
            pltpu.SemaphoreType.DMA,
        ],
    )(_sc_body)
    return f(x)
